# Initial kernel scaffold; baseline (speedup 1.0000x reference)
import jax
import jax.numpy as jnp
from jax.experimental import pallas as pl


def kernel(x, W):
    raise NotImplementedError("write your pallas kernel here")


# speedup vs baseline: 1.2749x; 1.2749x over previous
import functools

import jax
import jax.numpy as jnp
from jax.experimental import pallas as pl
from jax.experimental.pallas import tpu as pltpu

B, T, D = 4, 4096, 4096
E = 64
K = 8
COEF = 0.01
S = B * T
TS = 512
GRID = S // TS


def _router_body(x_ref, w_ref, idx_ref, gate_ref, zsq_ref, p_ref, c_ref,
                 aux_ref):
    i = pl.program_id(0)

    @pl.when(i == 0)
    def _init():
        zsq_ref[...] = jnp.zeros_like(zsq_ref)
        p_ref[...] = jnp.zeros_like(p_ref)
        c_ref[...] = jnp.zeros_like(c_ref)
        aux_ref[...] = jnp.zeros_like(aux_ref)

    logits = jax.lax.dot_general(
        x_ref[...], w_ref[...],
        dimension_numbers=(((1,), (1,)), ((), ())),
        preferred_element_type=jnp.float32,
        precision=jax.lax.Precision.DEFAULT)

    lane = jax.lax.broadcasted_iota(jnp.int32, (TS, E), 1)

    work = logits
    vals = []
    idxs = []
    for _ in range(K):
        m = jnp.max(work, axis=-1, keepdims=True)
        sel = jnp.min(jnp.where(work == m, lane, E), axis=-1,
                      keepdims=True)
        vals.append(m)
        idxs.append(sel)
        work = jnp.where(lane == sel, -jnp.inf, work)
    topv = jnp.concatenate(vals, axis=1)
    topi = jnp.concatenate(idxs, axis=1)

    idx_ref[...] = topi
    ex = jnp.exp(topv - topv[:, :1])
    gate_ref[...] = ex / jnp.sum(ex, axis=1, keepdims=True)

    m0 = topv[:, :1]
    pe = jnp.exp(logits - m0)
    se = jnp.sum(pe, axis=-1, keepdims=True)
    log_z = m0 + jnp.log(se)
    zsq_ref[...] += jnp.sum(log_z * log_z).reshape(1, 1)

    p_ref[...] += jnp.sum(pe / se, axis=0).reshape(1, E)
    c_ref[...] += jnp.sum(
        (topi[:, :1] == lane).astype(jnp.float32), axis=0).reshape(1, E)

    @pl.when(i == GRID - 1)
    def _fin():
        z_loss = COEF * (zsq_ref[...] / S)
        lb = COEF * E * jnp.sum((c_ref[...] / S) * (p_ref[...] / S))
        aux_ref[...] = z_loss + lb


@jax.jit
def kernel(x, W):
    x_flat = x.reshape(S, D)
    out_shapes = (
        jax.ShapeDtypeStruct((S, K), jnp.int32),
        jax.ShapeDtypeStruct((S, K), jnp.float32),
        jax.ShapeDtypeStruct((1, 1), jnp.float32),
        jax.ShapeDtypeStruct((1, E), jnp.float32),
        jax.ShapeDtypeStruct((1, E), jnp.float32),
        jax.ShapeDtypeStruct((1, 1), jnp.float32),
    )
    grid_spec = pl.GridSpec(
        grid=(GRID,),
        in_specs=[
            pl.BlockSpec((TS, D), lambda i: (i, 0)),
            pl.BlockSpec((E, D), lambda i: (0, 0)),
        ],
        out_specs=(
            pl.BlockSpec((TS, K), lambda i: (i, 0)),
            pl.BlockSpec((TS, K), lambda i: (i, 0)),
            pl.BlockSpec((1, 1), lambda i: (0, 0)),
            pl.BlockSpec((1, E), lambda i: (0, 0)),
            pl.BlockSpec((1, E), lambda i: (0, 0)),
            pl.BlockSpec((1, 1), lambda i: (0, 0)),
        ),
    )
    topi, gates, _, _, _, aux = pl.pallas_call(
        _router_body,
        grid_spec=grid_spec,
        out_shape=out_shapes,
        compiler_params=pltpu.CompilerParams(
            dimension_semantics=("arbitrary",)),
    )(x_flat, W)
    return topi, gates, aux[0, 0]


# speedup vs baseline: 1.3134x; 1.0302x over previous
import functools

import jax
import jax.numpy as jnp
from jax.experimental import pallas as pl
from jax.experimental.pallas import tpu as pltpu

B, T, D = 4, 4096, 4096
E = 64
K = 8
COEF = 0.01
S = B * T
TS = 1024
GRID = S // TS


def _router_body(x_ref, w_ref, idx_ref, gate_ref, zsq_ref, p_ref, c_ref,
                 aux_ref):
    i = pl.program_id(0)

    @pl.when(i == 0)
    def _init():
        zsq_ref[...] = jnp.zeros_like(zsq_ref)
        p_ref[...] = jnp.zeros_like(p_ref)
        c_ref[...] = jnp.zeros_like(c_ref)
        aux_ref[...] = jnp.zeros_like(aux_ref)

    logits = jax.lax.dot_general(
        x_ref[...], w_ref[...],
        dimension_numbers=(((1,), (1,)), ((), ())),
        preferred_element_type=jnp.float32,
        precision=jax.lax.Precision.DEFAULT)

    lane = jax.lax.broadcasted_iota(jnp.int32, (TS, E), 1)

    work = logits
    vals = []
    idxs = []
    for _ in range(K):
        m = jnp.max(work, axis=-1, keepdims=True)
        sel = jnp.min(jnp.where(work == m, lane, E), axis=-1,
                      keepdims=True)
        vals.append(m)
        idxs.append(sel)
        work = jnp.where(lane == sel, -jnp.inf, work)
    topv = jnp.concatenate(vals, axis=1)
    topi = jnp.concatenate(idxs, axis=1)

    idx_ref[...] = topi
    ex = jnp.exp(topv - topv[:, :1])
    gate_ref[...] = ex / jnp.sum(ex, axis=1, keepdims=True)

    m0 = topv[:, :1]
    pe = jnp.exp(logits - m0)
    se = jnp.sum(pe, axis=-1, keepdims=True)
    log_z = m0 + jnp.log(se)
    zsq_ref[...] += jnp.sum(log_z * log_z).reshape(1, 1)

    p_ref[...] += jnp.sum(pe / se, axis=0).reshape(1, E)
    c_ref[...] += jnp.sum(
        (topi[:, :1] == lane).astype(jnp.float32), axis=0).reshape(1, E)

    @pl.when(i == GRID - 1)
    def _fin():
        z_loss = COEF * (zsq_ref[...] / S)
        lb = COEF * E * jnp.sum((c_ref[...] / S) * (p_ref[...] / S))
        aux_ref[...] = z_loss + lb


@jax.jit
def kernel(x, W):
    x_flat = x.reshape(S, D)
    out_shapes = (
        jax.ShapeDtypeStruct((S, K), jnp.int32),
        jax.ShapeDtypeStruct((S, K), jnp.float32),
        jax.ShapeDtypeStruct((1, 1), jnp.float32),
        jax.ShapeDtypeStruct((1, E), jnp.float32),
        jax.ShapeDtypeStruct((1, E), jnp.float32),
        jax.ShapeDtypeStruct((1, 1), jnp.float32),
    )
    grid_spec = pl.GridSpec(
        grid=(GRID,),
        in_specs=[
            pl.BlockSpec((TS, D), lambda i: (i, 0)),
            pl.BlockSpec((E, D), lambda i: (0, 0)),
        ],
        out_specs=(
            pl.BlockSpec((TS, K), lambda i: (i, 0)),
            pl.BlockSpec((TS, K), lambda i: (i, 0)),
            pl.BlockSpec((1, 1), lambda i: (0, 0)),
            pl.BlockSpec((1, E), lambda i: (0, 0)),
            pl.BlockSpec((1, E), lambda i: (0, 0)),
            pl.BlockSpec((1, 1), lambda i: (0, 0)),
        ),
    )
    topi, gates, _, _, _, aux = pl.pallas_call(
        _router_body,
        grid_spec=grid_spec,
        out_shape=out_shapes,
        compiler_params=pltpu.CompilerParams(
            dimension_semantics=("arbitrary",)),
    )(x_flat, W)
    return topi, gates, aux[0, 0]
